# baseline (speedup 1.0000x reference)
import functools

import jax
import jax.numpy as jnp
from jax import lax
from jax.experimental import pallas as pl
from jax.experimental.pallas import tpu as pltpu
from jax.experimental.pallas import tpu_sc as plsc

VOCAB = 1000
DIM = 64
PDIM = 128
BATCH = 1024
SEQ = 50

NUM_CORES = 2
NUM_SUBCORES = 16
NUM_WORKERS = NUM_CORES * NUM_SUBCORES
ROWS_PER_W = BATCH // NUM_WORKERS
BLK = 4
NBLK = ROWS_PER_W // BLK


@functools.lru_cache(maxsize=1)
def _build():
    mesh = plsc.VectorSubcoreMesh(core_axis_name="c", subcore_axis_name="s")

    @functools.partial(
        pl.kernel,
        mesh=mesh,
        out_type=jax.ShapeDtypeStruct((BATCH, SEQ, DIM), jnp.float32),
        scratch_types=[
            pltpu.VMEM((ROWS_PER_W, SEQ), jnp.int32),
            pltpu.VMEM((56, PDIM), jnp.float32),
            pltpu.VMEM((BLK, SEQ, DIM), jnp.float32),
            pltpu.SemaphoreType.DMA,
        ],
    )
    def gather_kernel(idx_hbm, table_hbm, out_hbm, idx_v, gbuf, obuf, sem):
        wid = lax.axis_index("s") * NUM_CORES + lax.axis_index("c")
        base = wid * ROWS_PER_W
        pltpu.sync_copy(idx_hbm.at[pl.ds(base, ROWS_PER_W)], idx_v)

        @pl.loop(0, NBLK)
        def _(g):
            for b in range(BLK):
                pltpu.async_copy(
                    table_hbm.at[idx_v.at[g * BLK + b]],
                    gbuf.at[pl.ds(0, SEQ)],
                    sem,
                ).wait()

                @pl.loop(0, SEQ)
                def _(s):
                    for c in range(DIM // 16):
                        obuf[b, s, pl.ds(c * 16, 16)] = gbuf[s, pl.ds(c * 16, 16)]

            pltpu.sync_copy(obuf, out_hbm.at[pl.ds(base + g * BLK, BLK)])

    return gather_kernel


def kernel(token_ids, W):
    wp = jnp.pad(W, ((0, 0), (0, PDIM - DIM)))
    return _build()(token_ids.astype(jnp.int32), wp)


# speedup vs baseline: 1.3383x; 1.3383x over previous
import functools

import jax
import jax.numpy as jnp
from jax import lax
from jax.experimental import pallas as pl
from jax.experimental.pallas import tpu as pltpu
from jax.experimental.pallas import tpu_sc as plsc

VOCAB = 1000
DIM = 64
PDIM = 128
BATCH = 1024
SEQ = 50
TOKENS = BATCH * SEQ
PAIRS = TOKENS // 2

NUM_CORES = 2
NUM_SUBCORES = 16
NUM_WORKERS = NUM_CORES * NUM_SUBCORES
TOK_PER_W = TOKENS // NUM_WORKERS
PAIRS_PER_W = TOK_PER_W // 2
CP = 200
CT = 2 * CP
NCHUNK = PAIRS_PER_W // CP
GSUB = 80
NGSUB = CT // GSUB


@functools.lru_cache(maxsize=1)
def _build():
    mesh = plsc.VectorSubcoreMesh(core_axis_name="c", subcore_axis_name="s")

    @functools.partial(
        pl.kernel,
        mesh=mesh,
        out_type=jax.ShapeDtypeStruct((PAIRS, PDIM), jnp.float32),
        scratch_types=[
            pltpu.VMEM_SHARED((VOCAB, DIM), jnp.float32),
            pltpu.VMEM((TOK_PER_W,), jnp.int32),
            pltpu.VMEM((2, CT, DIM), jnp.float32),
            pltpu.VMEM((2, CP, PDIM), jnp.float32),
            pltpu.SemaphoreType.DMA,
            pltpu.SemaphoreType.DMA,
        ],
        compiler_params=pltpu.CompilerParams(use_tc_tiling_on_sc=False),
    )
    def gather_kernel(
        idx_hbm, table_hbm, out_hbm, table_s, idx_v, gbuf, wbuf, gsem, wsem
    ):
        sid = lax.axis_index("s")
        wid = sid * NUM_CORES + lax.axis_index("c")
        pbase = wid * PAIRS_PER_W

        @pl.when(sid == 0)
        def _():
            pltpu.sync_copy(table_hbm, table_s)

        pltpu.sync_copy(idx_hbm.at[pl.ds(wid * TOK_PER_W, TOK_PER_W)], idx_v)
        plsc.subcore_barrier()

        def fire_gathers(c):
            slot = c % 2
            return [
                pltpu.async_copy(
                    table_s.at[idx_v.at[pl.ds(c * CT + j * GSUB, GSUB)]],
                    gbuf.at[slot].at[pl.ds(j * GSUB, GSUB)],
                    gsem,
                )
                for j in range(NGSUB)
            ]

        def relay_and_write(c):
            slot = c % 2

            @pl.loop(0, CP)
            def _(p):
                for h in range(2):
                    for l in range(DIM // 16):
                        wbuf[slot, p, pl.ds(h * DIM + l * 16, 16)] = gbuf[
                            slot, 2 * p + h, pl.ds(l * 16, 16)
                        ]

            return pltpu.async_copy(
                wbuf.at[slot],
                out_hbm.at[pl.ds(pbase + c * CP, CP)],
                wsem,
            )

        writebacks = []
        pending = fire_gathers(0)
        for c in range(NCHUNK):
            nxt = fire_gathers(c + 1) if c + 1 < NCHUNK else []
            for g in pending:
                g.wait()
            if c >= 2:
                writebacks[c - 2].wait()
            writebacks.append(relay_and_write(c))
            pending = nxt
        for wb in writebacks[-2:]:
            wb.wait()

    return gather_kernel


def kernel(token_ids, W):
    idx = token_ids.reshape(-1).astype(jnp.int32)
    out2 = _build()(idx, W)
    return out2.reshape(BATCH, SEQ, DIM)


# speedup vs baseline: 1.5233x; 1.1382x over previous
import functools

import jax
import jax.numpy as jnp
from jax import lax
from jax.experimental import pallas as pl
from jax.experimental.pallas import tpu as pltpu
from jax.experimental.pallas import tpu_sc as plsc

VOCAB = 1000
DIM = 64
PDIM = 128
BATCH = 1024
SEQ = 50

NUM_CORES = 2
NUM_SUBCORES = 16
NUM_WORKERS = NUM_CORES * NUM_SUBCORES
ROWS_PER_W = BATCH // NUM_WORKERS
BLK = 4
NCHUNK = ROWS_PER_W // BLK
CT = BLK * SEQ


@functools.lru_cache(maxsize=1)
def _build():
    mesh = plsc.VectorSubcoreMesh(core_axis_name="c", subcore_axis_name="s")

    @functools.partial(
        pl.kernel,
        mesh=mesh,
        out_type=jax.ShapeDtypeStruct((BATCH, SEQ, DIM), jnp.float32),
        scratch_types=[
            pltpu.VMEM_SHARED((VOCAB, PDIM), jnp.float32),
            pltpu.VMEM((ROWS_PER_W, SEQ), jnp.int32),
            pltpu.VMEM((2, CT, PDIM), jnp.float32),
            pltpu.VMEM((2, BLK, SEQ, DIM), jnp.float32),
            pltpu.SemaphoreType.DMA,
            pltpu.SemaphoreType.DMA,
        ],
    )
    def gather_kernel(
        idx_hbm, table_hbm, out_hbm, table_s, idx_v, gbuf, obuf, gsem, wsem
    ):
        sid = lax.axis_index("s")
        wid = sid * NUM_CORES + lax.axis_index("c")
        base = wid * ROWS_PER_W

        @pl.when(sid == 0)
        def _():
            pltpu.sync_copy(table_hbm, table_s)

        pltpu.sync_copy(idx_hbm.at[pl.ds(base, ROWS_PER_W)], idx_v)
        plsc.subcore_barrier()

        def fire_gathers(c, slot):
            return [
                pltpu.async_copy(
                    table_s.at[idx_v.at[c * BLK + b]],
                    gbuf.at[slot].at[pl.ds(b * SEQ, SEQ)],
                    gsem,
                )
                for b in range(BLK)
            ]

        def relay(slot):
            for b in range(BLK):

                @pl.loop(0, SEQ)
                def _(s):
                    for l in range(DIM // 16):
                        obuf[slot, b, s, pl.ds(l * 16, 16)] = gbuf[
                            slot, b * SEQ + s, pl.ds(l * 16, 16)
                        ]

        def write_block(c, slot):
            return pltpu.async_copy(
                obuf.at[slot], out_hbm.at[pl.ds(base + c * BLK, BLK)], wsem
            )

        wbs = [None, None]

        @pl.loop(0, NCHUNK // 2)
        def _(i):
            c0 = 2 * i
            g0 = fire_gathers(c0, 0)
            g1 = fire_gathers(c0 + 1, 1)
            for g in g0:
                g.wait()
            relay(0)
            wb0 = write_block(c0, 0)
            for g in g1:
                g.wait()
            relay(1)
            wb1 = write_block(c0 + 1, 1)
            wb0.wait()
            wb1.wait()

    return gather_kernel


def kernel(token_ids, W):
    wp = jnp.pad(W, ((0, 0), (0, PDIM - DIM)))
    return _build()(token_ids.astype(jnp.int32), wp)
